# baseline (speedup 1.0000x reference)
import functools

import jax
import jax.numpy as jnp
from jax import lax
from jax.experimental import pallas as pl
from jax.experimental.pallas import tpu as pltpu
from jax.experimental.pallas import tpu_sc as plsc

N = 10000
NP = 10112
E = 320000
NC, NS = 2, 16
NW = NC * NS
EPW = E // NW
RPS = NP // NS


def _make_sc_agg(ch, nbuf, with_deg):
  d = 128
  nchunk = EPW // ch
  ngroup = nchunk // nbuf
  rem = nchunk - ngroup * nbuf
  mesh = plsc.VectorSubcoreMesh(core_axis_name="c", subcore_axis_name="s")
  acc_type = jax.ShapeDtypeStruct((NC, NP, d), jnp.float32)
  out_type = [acc_type, jax.ShapeDtypeStruct((NC, NP, 16), jnp.float32)
              ] if with_deg else acc_type
  scratch = [
      [pltpu.VMEM((nbuf, ch), jnp.int32) for _ in range(2)],
      [pltpu.VMEM((nbuf, ch), jnp.int32) for _ in range(2)],
      [pltpu.VMEM((ch, d), jnp.float32) for _ in range(nbuf)],
      pltpu.VMEM_SHARED((NP, d), jnp.float32),
      [pltpu.SemaphoreType.DMA for _ in range(nbuf)],
      [pltpu.SemaphoreType.DMA for _ in range(nbuf)],
      [pltpu.SemaphoreType.DMA for _ in range(2)],
  ]
  if with_deg:
    scratch.append(pltpu.VMEM((ch, 16), jnp.float32))
    scratch.append(pltpu.VMEM_SHARED((NP, 16), jnp.float32))

  @functools.partial(
      pl.kernel,
      out_type=out_type,
      mesh=mesh,
      compiler_params=pltpu.CompilerParams(use_tc_tiling_on_sc=False),
      scratch_types=scratch,
  )
  def agg(h_hbm, src_hbm, dst_hbm, zeros_hbm, *rest):
    if with_deg:
      (zeros16_hbm, out_hbm, deg_out, src_v, dst_v, rows, acc, gsem, ssem,
       isem, ones_v, dacc) = rest
    else:
      out_hbm, src_v, dst_v, rows, acc, gsem, ssem, isem = rest
    c = lax.axis_index("c")
    s = lax.axis_index("s")
    w = s * NC + c

    def stage_idx(g, p):
      pltpu.async_copy(src_hbm.at[w].at[pl.ds(g * nbuf, nbuf)], src_v[p],
                       isem[p])
      pltpu.async_copy(dst_hbm.at[w].at[pl.ds(g * nbuf, nbuf)], dst_v[p],
                       isem[p])

    def wait_idx(g, p):
      pltpu.make_async_copy(src_hbm.at[w].at[pl.ds(g * nbuf, nbuf)],
                            src_v[p], isem[p]).wait()
      pltpu.make_async_copy(dst_hbm.at[w].at[pl.ds(g * nbuf, nbuf)],
                            dst_v[p], isem[p]).wait()

    def issue_gather(b, p):
      pltpu.async_copy(h_hbm.at[src_v[p].at[b]], rows[b], gsem[b])

    def issue_scatter(b, p):
      pltpu.async_copy(rows[b], acc.at[dst_v[p].at[b]], ssem[b], add=True)
      if with_deg:
        pltpu.async_copy(ones_v, dacc.at[dst_v[p].at[b]], ssem[b], add=True)

    def wait_scatter(b, p):
      pltpu.make_async_copy(rows[b], acc.at[dst_v[p].at[b]], ssem[b]).wait()
      if with_deg:
        pltpu.make_async_copy(ones_v, dacc.at[dst_v[p].at[b]],
                              ssem[b]).wait()

    def step(g, p, nxt):
      if nxt:
        stage_idx(g + 1, 1 - p)
      for b in range(nbuf):
        pltpu.make_async_copy(h_hbm.at[src_v[p].at[b]], rows[b],
                              gsem[b]).wait()
        issue_scatter(b, p)
      if nxt:
        wait_idx(g + 1, 1 - p)
      for b in range(nbuf):
        wait_scatter(b, p)
        if nxt:
          issue_gather(b, 1 - p)

    pltpu.sync_copy(zeros_hbm.at[pl.ds(s * RPS, RPS)],
                    acc.at[pl.ds(s * RPS, RPS)])
    if with_deg:
      pltpu.sync_copy(zeros16_hbm.at[pl.ds(s * RPS, RPS)],
                      dacc.at[pl.ds(s * RPS, RPS)])

      def fill_ones(i, carry):
        ones_v[i, :] = jnp.ones((16,), jnp.float32)
        return carry

      lax.fori_loop(0, ch, fill_ones, 0)
    stage_idx(0, 0)
    wait_idx(0, 0)
    for b in range(nbuf):
      issue_gather(b, 0)
    plsc.subcore_barrier()

    def pair(k, carry):
      step(2 * k, 0, True)
      step(2 * k + 1, 1, True)
      return carry

    if ngroup % 2 == 0:
      lax.fori_loop(0, (ngroup - 2) // 2, pair, 0)
      step(ngroup - 2, 0, True)
      step(ngroup - 1, 1, False)
      free = 0
    else:
      lax.fori_loop(0, (ngroup - 3) // 2, pair, 0)
      step(ngroup - 3, 0, True)
      step(ngroup - 2, 1, True)
      step(ngroup - 1, 0, False)
      free = 1
    if rem:
      pltpu.sync_copy(src_hbm.at[w].at[pl.ds(ngroup * nbuf, rem)],
                      src_v[free].at[pl.ds(0, rem)])
      pltpu.sync_copy(dst_hbm.at[w].at[pl.ds(ngroup * nbuf, rem)],
                      dst_v[free].at[pl.ds(0, rem)])
      for b in range(rem):
        issue_gather(b, free)
      for b in range(rem):
        pltpu.make_async_copy(h_hbm.at[src_v[free].at[b]], rows[b],
                              gsem[b]).wait()
        issue_scatter(b, free)
      for b in range(rem):
        wait_scatter(b, free)
    plsc.subcore_barrier()
    pltpu.sync_copy(acc.at[pl.ds(s * RPS, RPS)],
                    out_hbm.at[c].at[pl.ds(s * RPS, RPS)])
    if with_deg:
      pltpu.sync_copy(dacc.at[pl.ds(s * RPS, RPS)],
                      deg_out.at[c].at[pl.ds(s * RPS, RPS)])

  return agg


_sc_agg_deg = _make_sc_agg(40, 7, True)
_sc_agg_128 = _make_sc_agg(80, 4, False)

_R = 1264


def _tc1_body(accp, degp, feat, w1, b1, h1_out, rinv_out):
  agg = accp[0] + accp[1]
  deg = degp[0, :, :1] + degp[1, :, :1]
  rinv = 1.0 / (deg + 1.0)
  hn = (agg + feat[...]) * rinv
  h1 = jnp.dot(hn, w1[...], preferred_element_type=jnp.float32) + b1[...]
  h1_out[...] = jnp.maximum(h1, 0.0)
  rinv_out[...] = jnp.broadcast_to(rinv, (_R, 16))


_tc1 = pl.pallas_call(
    _tc1_body,
    grid=(NP // _R,),
    in_specs=[
        pl.BlockSpec((NC, _R, 128), lambda i: (0, i, 0)),
        pl.BlockSpec((NC, _R, 16), lambda i: (0, i, 0)),
        pl.BlockSpec((_R, 128), lambda i: (i, 0)),
        pl.BlockSpec((128, 128), lambda i: (0, 0)),
        pl.BlockSpec((1, 128), lambda i: (0, 0)),
    ],
    out_specs=[
        pl.BlockSpec((_R, 128), lambda i: (i, 0)),
        pl.BlockSpec((_R, 16), lambda i: (i, 0)),
    ],
    out_shape=[
        jax.ShapeDtypeStruct((NP, 128), jnp.float32),
        jax.ShapeDtypeStruct((NP, 16), jnp.float32),
    ],
)


def _leaky(x):
  return jnp.where(x > 0, x, 0.01 * x)


def _tc2_body(accp, h1, rinv, w2, b2, d1, bd1, d2, bd2, out):
  hsum = accp[0] + accp[1]
  hn = (hsum + h1[...]) * rinv[:, :1]
  h2 = jnp.dot(hn, w2[...], preferred_element_type=jnp.float32) + b2[...]
  h2 = jnp.maximum(h2, 0.0)
  t = jnp.dot(h2, d1[...], preferred_element_type=jnp.float32) + bd1[...]
  t = _leaky(t)
  o = jnp.dot(t, d2[...], preferred_element_type=jnp.float32) + bd2[...]
  out[...] = _leaky(o)


_tc2 = pl.pallas_call(
    _tc2_body,
    grid=(NP // _R,),
    in_specs=[
        pl.BlockSpec((NC, _R, 128), lambda i: (0, i, 0)),
        pl.BlockSpec((_R, 128), lambda i: (i, 0)),
        pl.BlockSpec((_R, 16), lambda i: (i, 0)),
        pl.BlockSpec((128, 128), lambda i: (0, 0)),
        pl.BlockSpec((1, 128), lambda i: (0, 0)),
        pl.BlockSpec((128, 256), lambda i: (0, 0)),
        pl.BlockSpec((1, 256), lambda i: (0, 0)),
        pl.BlockSpec((256, 40), lambda i: (0, 0)),
        pl.BlockSpec((1, 40), lambda i: (0, 0)),
    ],
    out_specs=pl.BlockSpec((_R, 40), lambda i: (i, 0)),
    out_shape=jax.ShapeDtypeStruct((NP, 40), jnp.float32),
)


@jax.jit
def kernel(features, edge_index, W1, b1, W2, b2, D1, bd1, D2, bd2):
  ei = edge_index.astype(jnp.int32)
  src40 = ei[0].reshape(NW, EPW // 40, 40)
  dst40 = ei[1].reshape(NW, EPW // 40, 40)
  src80 = ei[0].reshape(NW, EPW // 80, 80)
  dst80 = ei[1].reshape(NW, EPW // 80, 80)
  featp = jnp.pad(features, ((0, NP - N), (0, 0)))
  z128 = jnp.zeros((NP, 128), jnp.float32)
  acc1, deg1 = _sc_agg_deg(featp, src40, dst40, z128,
                           jnp.zeros((NP, 16), jnp.float32))
  h1, rinv = _tc1(acc1, deg1, featp, W1, b1.reshape(1, 128))
  acc2 = _sc_agg_128(h1, src80, dst80, z128)
  out = _tc2(acc2, h1, rinv, W2, b2.reshape(1, 128),
             D1, bd1.reshape(1, 256), D2, bd2.reshape(1, 40))
  return out[:N]


# speedup vs baseline: 1.0050x; 1.0050x over previous
import functools

import jax
import jax.numpy as jnp
from jax import lax
from jax.experimental import pallas as pl
from jax.experimental.pallas import tpu as pltpu
from jax.experimental.pallas import tpu_sc as plsc

N = 10000
NP = 10112
E = 320000
NC, NS = 2, 16
NW = NC * NS
EPW = E // NW
RPS = NP // NS


def _make_sc_agg(ch, nbuf, with_deg):
  d = 128
  nchunk = EPW // ch
  ngroup = nchunk // nbuf
  rem = nchunk - ngroup * nbuf
  mesh = plsc.VectorSubcoreMesh(core_axis_name="c", subcore_axis_name="s")
  acc_type = jax.ShapeDtypeStruct((NC, NP, d), jnp.float32)
  out_type = [acc_type, jax.ShapeDtypeStruct((NC, NP, 16), jnp.float32)
              ] if with_deg else acc_type
  scratch = [
      [pltpu.VMEM((nbuf, ch), jnp.int32) for _ in range(2)],
      [pltpu.VMEM((nbuf, ch), jnp.int32) for _ in range(2)],
      [pltpu.VMEM((ch, d), jnp.float32) for _ in range(nbuf)],
      pltpu.VMEM_SHARED((NP, d), jnp.float32),
      [pltpu.SemaphoreType.DMA for _ in range(nbuf)],
      [pltpu.SemaphoreType.DMA for _ in range(nbuf)],
      [pltpu.SemaphoreType.DMA for _ in range(2)],
  ]
  if with_deg:
    scratch.append(pltpu.VMEM((ch, 16), jnp.float32))
    scratch.append(pltpu.VMEM_SHARED((NP, 16), jnp.float32))

  @functools.partial(
      pl.kernel,
      out_type=out_type,
      mesh=mesh,
      compiler_params=pltpu.CompilerParams(use_tc_tiling_on_sc=False),
      scratch_types=scratch,
  )
  def agg(h_hbm, src_hbm, dst_hbm, zeros_hbm, *rest):
    if with_deg:
      (zeros16_hbm, out_hbm, deg_out, src_v, dst_v, rows, acc, gsem, ssem,
       isem, ones_v, dacc) = rest
    else:
      out_hbm, src_v, dst_v, rows, acc, gsem, ssem, isem = rest
    c = lax.axis_index("c")
    s = lax.axis_index("s")
    w = s * NC + c

    def stage_idx(g, p):
      pltpu.async_copy(src_hbm.at[w].at[pl.ds(g * nbuf, nbuf)], src_v[p],
                       isem[p])
      pltpu.async_copy(dst_hbm.at[w].at[pl.ds(g * nbuf, nbuf)], dst_v[p],
                       isem[p])

    def wait_idx(g, p):
      pltpu.make_async_copy(src_hbm.at[w].at[pl.ds(g * nbuf, nbuf)],
                            src_v[p], isem[p]).wait()
      pltpu.make_async_copy(dst_hbm.at[w].at[pl.ds(g * nbuf, nbuf)],
                            dst_v[p], isem[p]).wait()

    def issue_gather(b, p):
      pltpu.async_copy(h_hbm.at[src_v[p].at[b]], rows[b], gsem[b])

    def issue_scatter(b, p):
      pltpu.async_copy(rows[b], acc.at[dst_v[p].at[b]], ssem[b], add=True)
      if with_deg:
        pltpu.async_copy(ones_v, dacc.at[dst_v[p].at[b]], ssem[b], add=True)

    def wait_scatter(b, p):
      pltpu.make_async_copy(rows[b], acc.at[dst_v[p].at[b]], ssem[b]).wait()
      if with_deg:
        pltpu.make_async_copy(ones_v, dacc.at[dst_v[p].at[b]],
                              ssem[b]).wait()

    def step(g, p, nxt):
      if nxt:
        stage_idx(g + 1, 1 - p)
      for b in range(nbuf):
        pltpu.make_async_copy(h_hbm.at[src_v[p].at[b]], rows[b],
                              gsem[b]).wait()
        issue_scatter(b, p)
      if nxt:
        wait_idx(g + 1, 1 - p)
      for b in range(nbuf):
        wait_scatter(b, p)
        if nxt:
          issue_gather(b, 1 - p)

    pltpu.sync_copy(zeros_hbm.at[pl.ds(s * RPS, RPS)],
                    acc.at[pl.ds(s * RPS, RPS)])
    if with_deg:
      pltpu.sync_copy(zeros16_hbm.at[pl.ds(s * RPS, RPS)],
                      dacc.at[pl.ds(s * RPS, RPS)])

      def fill_ones(i, carry):
        ones_v[i, :] = jnp.ones((16,), jnp.float32)
        return carry

      lax.fori_loop(0, ch, fill_ones, 0)
    stage_idx(0, 0)
    wait_idx(0, 0)
    for b in range(nbuf):
      issue_gather(b, 0)
    plsc.subcore_barrier()

    def pair(k, carry):
      step(2 * k, 0, True)
      step(2 * k + 1, 1, True)
      return carry

    if ngroup % 2 == 0:
      lax.fori_loop(0, (ngroup - 2) // 2, pair, 0)
      step(ngroup - 2, 0, True)
      step(ngroup - 1, 1, False)
      free = 0
    else:
      lax.fori_loop(0, (ngroup - 3) // 2, pair, 0)
      step(ngroup - 3, 0, True)
      step(ngroup - 2, 1, True)
      step(ngroup - 1, 0, False)
      free = 1
    if rem:
      pltpu.sync_copy(src_hbm.at[w].at[pl.ds(ngroup * nbuf, rem)],
                      src_v[free].at[pl.ds(0, rem)])
      pltpu.sync_copy(dst_hbm.at[w].at[pl.ds(ngroup * nbuf, rem)],
                      dst_v[free].at[pl.ds(0, rem)])
      for b in range(rem):
        issue_gather(b, free)
      for b in range(rem):
        pltpu.make_async_copy(h_hbm.at[src_v[free].at[b]], rows[b],
                              gsem[b]).wait()
        issue_scatter(b, free)
      for b in range(rem):
        wait_scatter(b, free)
    plsc.subcore_barrier()
    pltpu.sync_copy(acc.at[pl.ds(s * RPS, RPS)],
                    out_hbm.at[c].at[pl.ds(s * RPS, RPS)])
    if with_deg:
      pltpu.sync_copy(dacc.at[pl.ds(s * RPS, RPS)],
                      deg_out.at[c].at[pl.ds(s * RPS, RPS)])

  return agg


_sc_agg_deg = _make_sc_agg(40, 7, True)
_sc_agg_128 = _make_sc_agg(40, 9, False)

_R = 1264


def _tc1_body(accp, degp, feat, w1, b1, h1_out, rinv_out):
  agg = accp[0] + accp[1]
  deg = degp[0, :, :1] + degp[1, :, :1]
  rinv = 1.0 / (deg + 1.0)
  hn = (agg + feat[...]) * rinv
  h1 = jnp.dot(hn, w1[...], preferred_element_type=jnp.float32) + b1[...]
  h1_out[...] = jnp.maximum(h1, 0.0)
  rinv_out[...] = jnp.broadcast_to(rinv, (_R, 16))


_tc1 = pl.pallas_call(
    _tc1_body,
    grid=(NP // _R,),
    in_specs=[
        pl.BlockSpec((NC, _R, 128), lambda i: (0, i, 0)),
        pl.BlockSpec((NC, _R, 16), lambda i: (0, i, 0)),
        pl.BlockSpec((_R, 128), lambda i: (i, 0)),
        pl.BlockSpec((128, 128), lambda i: (0, 0)),
        pl.BlockSpec((1, 128), lambda i: (0, 0)),
    ],
    out_specs=[
        pl.BlockSpec((_R, 128), lambda i: (i, 0)),
        pl.BlockSpec((_R, 16), lambda i: (i, 0)),
    ],
    out_shape=[
        jax.ShapeDtypeStruct((NP, 128), jnp.float32),
        jax.ShapeDtypeStruct((NP, 16), jnp.float32),
    ],
)


def _leaky(x):
  return jnp.where(x > 0, x, 0.01 * x)


def _tc2_body(accp, h1, rinv, w2, b2, d1, bd1, d2, bd2, out):
  hsum = accp[0] + accp[1]
  hn = (hsum + h1[...]) * rinv[:, :1]
  h2 = jnp.dot(hn, w2[...], preferred_element_type=jnp.float32) + b2[...]
  h2 = jnp.maximum(h2, 0.0)
  t = jnp.dot(h2, d1[...], preferred_element_type=jnp.float32) + bd1[...]
  t = _leaky(t)
  o = jnp.dot(t, d2[...], preferred_element_type=jnp.float32) + bd2[...]
  out[...] = _leaky(o)


_tc2 = pl.pallas_call(
    _tc2_body,
    grid=(NP // _R,),
    in_specs=[
        pl.BlockSpec((NC, _R, 128), lambda i: (0, i, 0)),
        pl.BlockSpec((_R, 128), lambda i: (i, 0)),
        pl.BlockSpec((_R, 16), lambda i: (i, 0)),
        pl.BlockSpec((128, 128), lambda i: (0, 0)),
        pl.BlockSpec((1, 128), lambda i: (0, 0)),
        pl.BlockSpec((128, 256), lambda i: (0, 0)),
        pl.BlockSpec((1, 256), lambda i: (0, 0)),
        pl.BlockSpec((256, 40), lambda i: (0, 0)),
        pl.BlockSpec((1, 40), lambda i: (0, 0)),
    ],
    out_specs=pl.BlockSpec((_R, 40), lambda i: (i, 0)),
    out_shape=jax.ShapeDtypeStruct((NP, 40), jnp.float32),
)


@jax.jit
def kernel(features, edge_index, W1, b1, W2, b2, D1, bd1, D2, bd2):
  ei = edge_index.astype(jnp.int32)
  src40 = ei[0].reshape(NW, EPW // 40, 40)
  dst40 = ei[1].reshape(NW, EPW // 40, 40)
  featp = jnp.pad(features, ((0, NP - N), (0, 0)))
  z128 = jnp.zeros((NP, 128), jnp.float32)
  acc1, deg1 = _sc_agg_deg(featp, src40, dst40, z128,
                           jnp.zeros((NP, 16), jnp.float32))
  h1, rinv = _tc1(acc1, deg1, featp, W1, b1.reshape(1, 128))
  acc2 = _sc_agg_128(h1, src40, dst40, z128)
  out = _tc2(acc2, h1, rinv, W2, b2.reshape(1, 128),
             D1, bd1.reshape(1, 256), D2, bd2.reshape(1, 40))
  return out[:N]
